# baseline (speedup 1.0000x reference)
import functools

import jax
import jax.numpy as jnp
from jax import lax
from jax.experimental import pallas as pl
from jax.experimental.pallas import tpu as pltpu
from jax.experimental.pallas import tpu_sc as plsc

BATCH = 16384
EMBED = 32
ROWS = 1000000
NUM_CORES = 2
NUM_SUBCORES = 16
LANES = 16
NUM_WORKERS = NUM_CORES * NUM_SUBCORES
BPW = BATCH // NUM_WORKERS
TCOLS = (ROWS + 127) // 128
TPW = (TCOLS + NUM_WORKERS - 1) // NUM_WORKERS
NBUCK = 2 * TPW
CAP = 2048
DEPTH = 12
STAG = 32


def _extract_body(u_tab, i_tab, u_idx_hbm, i_idx_hbm, n_idx_hbm,
                  emb,
                  vidx, mly_r, mly_id, srt_r, srt_id, ring, stag,
                  starts, cursor,
                  sem_slab, sem_out):
    wid = lax.axis_index("s") * NUM_CORES + lax.axis_index("c")
    lo = wid * TPW
    hi = jnp.minimum(lo + TPW, TCOLS)

    idx_hbms = (u_idx_hbm, i_idx_hbm, n_idx_hbm)
    lane_iota = lax.iota(jnp.int32, LANES)

    for t in range(3):
        pltpu.sync_copy(idx_hbms[t], vidx.at[pl.ds(t * BATCH, BATCH)])

    def compact_body(i, off):
        newoff = off
        for t in range(3):
            v = vidx[pl.ds(t * BATCH + i * LANES, LANES)]
            tc = lax.shift_right_logical(v, 7)
            mine = (tc >= lo) & (tc < hi)
            pc = jnp.cumsum(mine.astype(jnp.int32))
            pos = newoff + pc - 1
            k = t * BATCH + i * LANES + lane_iota
            plsc.store_scatter(mly_r, [pos], v, mask=mine)
            plsc.store_scatter(mly_id, [pos], k, mask=mine)
            newoff = newoff + pc[LANES - 1]
        return newoff
    nrec = lax.fori_loop(0, BATCH // LANES, compact_body, jnp.int32(0))

    def zero_body(b, _):
        cursor[b] = jnp.int32(0)
        return ()
    lax.fori_loop(0, NBUCK, zero_body, ())

    def hist_body(q, _):
        rv = mly_r[pl.ds(q * LANES, LANES)]
        iv = mly_id[pl.ds(q * LANES, LANES)]
        rem = nrec - q * LANES
        bv = jnp.clip((lax.shift_right_logical(rv, 7) - lo) * 2
                      + jnp.minimum(lax.shift_right_logical(iv, 14), 1),
                      0, NBUCK - 1)
        for lane in range(LANES):
            @pl.when(rem > lane)
            def _():
                b = bv[lane]
                cursor[b] = cursor[b] + 1
        return ()
    lax.fori_loop(0, (nrec + LANES - 1) // LANES, hist_body, ())

    def scan_body(b, acc):
        c = cursor[b]
        starts[b] = acc
        cursor[b] = acc
        return acc + c
    total = lax.fori_loop(0, NBUCK, scan_body, jnp.int32(0))
    starts[NBUCK] = total

    def place_body(q, _):
        rv = mly_r[pl.ds(q * LANES, LANES)]
        iv = mly_id[pl.ds(q * LANES, LANES)]
        rem = nrec - q * LANES
        bv = jnp.clip((lax.shift_right_logical(rv, 7) - lo) * 2
                      + jnp.minimum(lax.shift_right_logical(iv, 14), 1),
                      0, NBUCK - 1)
        mask = lane_iota < rem
        pos = jnp.zeros((LANES,), jnp.int32)
        for lane in range(LANES):
            b = bv[lane]
            ps = cursor[b]
            pos = jnp.where(lane_iota == lane, ps, pos)
            @pl.when(rem > lane)
            def _():
                cursor[b] = ps + 1
        plsc.store_scatter(srt_r, [pos], rv, mask=mask)
        plsc.store_scatter(srt_id, [pos], iv, mask=mask)
        return ()
    lax.fori_loop(0, (nrec + LANES - 1) // LANES, place_body, ())

    rows_lo = lane_iota
    rows_hi = lane_iota + LANES

    def issue_slab(b):
        cnt = starts[b + 1] - starts[b]
        col = lo + lax.shift_right_logical(b, 1)
        c0 = pl.multiple_of(col * 128, 128)
        @pl.when((cnt > 0) & (b % 2 == 0))
        def _():
            pltpu.async_copy(u_tab.at[:, pl.ds(c0, 128)],
                             ring.at[b % DEPTH], sem_slab)
        @pl.when((cnt > 0) & (b % 2 == 1))
        def _():
            pltpu.async_copy(i_tab.at[:, pl.ds(c0, 128)],
                             ring.at[b % DEPTH], sem_slab)

    def drain_slab():
        pltpu.make_async_copy(u_tab.at[:, pl.ds(0, 128)],
                              ring.at[0], sem_slab).wait()

    def process_bucket(p, n_out):
        beg = starts[p]
        cnt = starts[p + 1] - beg
        slot = p % DEPTH

        def rec_body(q, n_out):
            off = beg + q * LANES
            rv = srt_r[pl.ds(off, LANES)]
            iv = srt_id[pl.ds(off, LANES)]
            rem = cnt - q * LANES
            for lane in range(LANES):
                valid = rem > lane
                so = n_out % STAG
                @pl.when(valid)
                def _():
                    c = jnp.full((LANES,), rv[lane] & 127, jnp.int32)
                    v0 = plsc.load_gather(ring.at[slot], [rows_lo, c])
                    v1 = plsc.load_gather(ring.at[slot], [rows_hi, c])
                    @pl.when(n_out >= STAG)
                    def _():
                        pltpu.make_async_copy(
                            emb.at[pl.ds(0, EMBED)], stag.at[0],
                            sem_out).wait()
                    stag[so, pl.ds(0, LANES)] = v0
                    stag[so, pl.ds(LANES, LANES)] = v1
                    pltpu.async_copy(
                        stag.at[so],
                        emb.at[pl.ds(iv[lane] * EMBED, EMBED)],
                        sem_out)
                n_out = n_out + jnp.where(valid, 1, 0)
            return n_out

        nq = (cnt + LANES - 1) // LANES
        return lax.fori_loop(0, nq, rec_body, n_out)

    def maybe_process(p, n_out):
        @pl.when(starts[p + 1] - starts[p] > 0)
        def _():
            drain_slab()
        return lax.cond(starts[p + 1] - starts[p] > 0,
                        lambda n: process_bucket(p, n),
                        lambda n: n,
                        n_out)

    def bucket_body(b, n_out):
        issue_slab(b)
        p = b - (DEPTH - 1)
        return lax.cond(p >= 0,
                        lambda n: maybe_process(jnp.maximum(p, 0), n),
                        lambda n: n,
                        n_out)

    n_out = lax.fori_loop(0, NBUCK, bucket_body, jnp.int32(0))

    def tail_body(b0, n_out):
        return maybe_process(NBUCK - (DEPTH - 1) + b0, n_out)
    n_out = lax.fori_loop(0, DEPTH - 1, tail_body, n_out)

    def outdrain_body(i, _):
        pltpu.make_async_copy(emb.at[pl.ds(0, EMBED)], stag.at[0],
                              sem_out).wait()
        return ()
    lax.fori_loop(0, jnp.minimum(n_out, STAG), outdrain_body, ())


def _compute_body(emb, out_hbm, uv, iv, nv, out_v, sem):
    wid = lax.axis_index("s") * NUM_CORES + lax.axis_index("c")
    base = wid * BPW
    pltpu.sync_copy(emb.at[pl.ds(base * EMBED, BPW * EMBED)], uv)
    pltpu.sync_copy(emb.at[pl.ds((BATCH + base) * EMBED, BPW * EMBED)], iv)
    pltpu.sync_copy(emb.at[pl.ds((2 * BATCH + base) * EMBED, BPW * EMBED)], nv)

    lane_iota = lax.iota(jnp.int32, LANES)

    def group_body(g, _):
        acc = jnp.zeros((LANES,), jnp.float32)
        for j in range(LANES):
            w = (g * LANES + j) * EMBED
            u0 = uv[pl.ds(w, LANES)]
            u1 = uv[pl.ds(w + LANES, LANES)]
            i0 = iv[pl.ds(w, LANES)]
            i1 = iv[pl.ds(w + LANES, LANES)]
            n0 = nv[pl.ds(w, LANES)]
            n1 = nv[pl.ds(w + LANES, LANES)]
            tv = u0 * (i0 - n0) + u1 * (i1 - n1)
            acc = jnp.where(lane_iota == j, jnp.sum(tv), acc)
        prob = 1.0 / (1.0 + jnp.exp(-acc))
        out_v[pl.ds(g * LANES, LANES)] = prob
        return ()

    lax.fori_loop(0, BPW // LANES, group_body, ())
    pltpu.sync_copy(out_v, out_hbm.at[pl.ds(base, BPW)])


@jax.jit
def kernel(user_table, item_table, user_tensor, item_tensor, nega_item_tensor):
    mesh = plsc.VectorSubcoreMesh(core_axis_name="c", subcore_axis_name="s")
    params = pltpu.CompilerParams(
        needs_layout_passes=False, use_tc_tiling_on_sc=True,
        disable_bounds_checks=True)

    extract = pl.kernel(
        _extract_body,
        out_type=jax.ShapeDtypeStruct((3 * BATCH * EMBED,), jnp.float32),
        mesh=mesh,
        scratch_types=[
            pltpu.VMEM((3 * BATCH,), jnp.int32),
            pltpu.VMEM((CAP,), jnp.int32),
            pltpu.VMEM((CAP,), jnp.int32),
            pltpu.VMEM((CAP,), jnp.int32),
            pltpu.VMEM((CAP,), jnp.int32),
            pltpu.VMEM((DEPTH, EMBED, 128), jnp.float32),
            pltpu.VMEM((STAG, EMBED), jnp.float32),
            pltpu.SMEM((NBUCK + 1,), jnp.int32),
            pltpu.SMEM((NBUCK,), jnp.int32),
            pltpu.SemaphoreType.DMA,
            pltpu.SemaphoreType.DMA,
        ],
        compiler_params=params,
    )
    compute = pl.kernel(
        _compute_body,
        out_type=jax.ShapeDtypeStruct((BATCH,), jnp.float32),
        mesh=mesh,
        scratch_types=[
            pltpu.VMEM((BPW * EMBED,), jnp.float32),
            pltpu.VMEM((BPW * EMBED,), jnp.float32),
            pltpu.VMEM((BPW * EMBED,), jnp.float32),
            pltpu.VMEM((BPW,), jnp.float32),
            pltpu.SemaphoreType.DMA,
        ],
        compiler_params=params,
    )

    emb = extract(
        user_table.T,
        item_table.T,
        user_tensor.astype(jnp.int32),
        item_tensor.astype(jnp.int32),
        nega_item_tensor.astype(jnp.int32),
    )
    return compute(emb)


# speedup vs baseline: 1.0060x; 1.0060x over previous
import functools

import jax
import jax.numpy as jnp
from jax import lax
from jax.experimental import pallas as pl
from jax.experimental.pallas import tpu as pltpu
from jax.experimental.pallas import tpu_sc as plsc

BATCH = 16384
EMBED = 32
ROWS = 1000000
NUM_CORES = 2
NUM_SUBCORES = 16
LANES = 16
NUM_WORKERS = NUM_CORES * NUM_SUBCORES
BPW = BATCH // NUM_WORKERS
TCOLS = (ROWS + 127) // 128
TPW = (TCOLS + NUM_WORKERS - 1) // NUM_WORKERS
NBUCK = 2 * TPW
RCAP = 768
DEPTH = 16
STAG = 32


def _extract_body(u_tab, i_tab, u_idx_hbm, i_idx_hbm, n_idx_hbm,
                  emb,
                  vidx, mly_r, mly_id, srt_r, srt_id, ring, stag,
                  starts, cursor,
                  sem_slab, sem_out):
    wid = lax.axis_index("s") * NUM_CORES + lax.axis_index("c")
    lo = wid * TPW
    hi = jnp.minimum(lo + TPW, TCOLS)

    idx_hbms = (u_idx_hbm, i_idx_hbm, n_idx_hbm)
    lane_iota = lax.iota(jnp.int32, LANES)

    for t in range(3):
        pltpu.sync_copy(idx_hbms[t], vidx.at[pl.ds(t * BATCH, BATCH)])

    def compact_body(i, offs):
        newoffs = []
        for t in range(3):
            off = offs[t]
            v = vidx[pl.ds(t * BATCH + i * LANES, LANES)]
            tc = lax.shift_right_logical(v, 7)
            mine = (tc >= lo) & (tc < hi)
            pc = jnp.cumsum(mine.astype(jnp.int32))
            pos = off + pc - 1
            k = t * BATCH + i * LANES + lane_iota
            plsc.store_scatter(mly_r, [pos], v, mask=mine)
            plsc.store_scatter(mly_id, [pos], k, mask=mine)
            newoffs.append(off + pc[LANES - 1])
        return tuple(newoffs)
    offs = lax.fori_loop(0, BATCH // LANES, compact_body,
                         tuple(jnp.int32(t * RCAP) for t in range(3)))
    nrecs = tuple(offs[t] - t * RCAP for t in range(3))

    def zero_body(b, _):
        cursor[b] = jnp.int32(0)
        return ()
    lax.fori_loop(0, NBUCK, zero_body, ())

    def hist_body(q, _, base=0, cnt=None):
        rv = mly_r[pl.ds(base + q * LANES, LANES)]
        iv = mly_id[pl.ds(base + q * LANES, LANES)]
        rem = cnt - q * LANES
        bv = jnp.clip((lax.shift_right_logical(rv, 7) - lo) * 2
                      + jnp.minimum(lax.shift_right_logical(iv, 14), 1),
                      0, NBUCK - 1)
        for lane in range(LANES):
            @pl.when(rem > lane)
            def _():
                b = bv[lane]
                cursor[b] = cursor[b] + 1
        return ()
    for t in range(3):
        lax.fori_loop(0, (nrecs[t] + LANES - 1) // LANES,
                      functools.partial(hist_body, base=t * RCAP,
                                        cnt=nrecs[t]), ())

    def scan_body(b, acc):
        c = cursor[b]
        starts[b] = acc
        cursor[b] = acc
        return acc + c
    total = lax.fori_loop(0, NBUCK, scan_body, jnp.int32(0))
    starts[NBUCK] = total

    def place_body(q, _, base=0, cnt=None):
        rv = mly_r[pl.ds(base + q * LANES, LANES)]
        iv = mly_id[pl.ds(base + q * LANES, LANES)]
        rem = cnt - q * LANES
        bv = jnp.clip((lax.shift_right_logical(rv, 7) - lo) * 2
                      + jnp.minimum(lax.shift_right_logical(iv, 14), 1),
                      0, NBUCK - 1)
        mask = lane_iota < rem
        pos = jnp.zeros((LANES,), jnp.int32)
        for lane in range(LANES):
            b = bv[lane]
            ps = cursor[b]
            pos = jnp.where(lane_iota == lane, ps, pos)
            @pl.when(rem > lane)
            def _():
                cursor[b] = ps + 1
        plsc.store_scatter(srt_r, [pos], rv, mask=mask)
        plsc.store_scatter(srt_id, [pos], iv, mask=mask)
        return ()
    for t in range(3):
        lax.fori_loop(0, (nrecs[t] + LANES - 1) // LANES,
                      functools.partial(place_body, base=t * RCAP,
                                        cnt=nrecs[t]), ())

    rows_lo = lane_iota
    rows_hi = lane_iota + LANES

    def issue_slab(b):
        cnt = starts[b + 1] - starts[b]
        col = lo + lax.shift_right_logical(b, 1)
        c0 = pl.multiple_of(col * 128, 128)
        @pl.when((cnt > 0) & (b % 2 == 0))
        def _():
            pltpu.async_copy(u_tab.at[:, pl.ds(c0, 128)],
                             ring.at[b % DEPTH], sem_slab)
        @pl.when((cnt > 0) & (b % 2 == 1))
        def _():
            pltpu.async_copy(i_tab.at[:, pl.ds(c0, 128)],
                             ring.at[b % DEPTH], sem_slab)

    def drain_slab():
        pltpu.make_async_copy(u_tab.at[:, pl.ds(0, 128)],
                              ring.at[0], sem_slab).wait()

    def process_bucket(p, n_out):
        beg = starts[p]
        cnt = starts[p + 1] - beg
        slot = p % DEPTH

        def rec_body(q, n_out):
            off = beg + q * LANES
            rv = srt_r[pl.ds(off, LANES)]
            iv = srt_id[pl.ds(off, LANES)]
            rem = cnt - q * LANES
            for lane in range(LANES):
                valid = rem > lane
                so = n_out % STAG
                @pl.when(valid)
                def _():
                    c = jnp.full((LANES,), rv[lane] & 127, jnp.int32)
                    v0 = plsc.load_gather(ring.at[slot], [rows_lo, c])
                    v1 = plsc.load_gather(ring.at[slot], [rows_hi, c])
                    @pl.when(n_out >= STAG)
                    def _():
                        pltpu.make_async_copy(
                            emb.at[pl.ds(0, EMBED)], stag.at[0],
                            sem_out).wait()
                    stag[so, pl.ds(0, LANES)] = v0
                    stag[so, pl.ds(LANES, LANES)] = v1
                    pltpu.async_copy(
                        stag.at[so],
                        emb.at[pl.ds(iv[lane] * EMBED, EMBED)],
                        sem_out)
                n_out = n_out + jnp.where(valid, 1, 0)
            return n_out

        nq = (cnt + LANES - 1) // LANES
        return lax.fori_loop(0, nq, rec_body, n_out)

    def maybe_process(p, n_out):
        @pl.when(starts[p + 1] - starts[p] > 0)
        def _():
            drain_slab()
        return lax.cond(starts[p + 1] - starts[p] > 0,
                        lambda n: process_bucket(p, n),
                        lambda n: n,
                        n_out)

    def bucket_body(b, n_out):
        issue_slab(b)
        p = b - (DEPTH - 1)
        return lax.cond(p >= 0,
                        lambda n: maybe_process(jnp.maximum(p, 0), n),
                        lambda n: n,
                        n_out)

    n_out = lax.fori_loop(0, NBUCK, bucket_body, jnp.int32(0))

    def tail_body(b0, n_out):
        return maybe_process(NBUCK - (DEPTH - 1) + b0, n_out)
    n_out = lax.fori_loop(0, DEPTH - 1, tail_body, n_out)

    def outdrain_body(i, _):
        pltpu.make_async_copy(emb.at[pl.ds(0, EMBED)], stag.at[0],
                              sem_out).wait()
        return ()
    lax.fori_loop(0, jnp.minimum(n_out, STAG), outdrain_body, ())


def _compute_body(emb, out_hbm, uv, iv, nv, out_v, sem):
    wid = lax.axis_index("s") * NUM_CORES + lax.axis_index("c")
    base = wid * BPW
    pltpu.sync_copy(emb.at[pl.ds(base * EMBED, BPW * EMBED)], uv)
    pltpu.sync_copy(emb.at[pl.ds((BATCH + base) * EMBED, BPW * EMBED)], iv)
    pltpu.sync_copy(emb.at[pl.ds((2 * BATCH + base) * EMBED, BPW * EMBED)], nv)

    lane_iota = lax.iota(jnp.int32, LANES)

    def group_body(g, _):
        acc = jnp.zeros((LANES,), jnp.float32)
        for j in range(LANES):
            w = (g * LANES + j) * EMBED
            u0 = uv[pl.ds(w, LANES)]
            u1 = uv[pl.ds(w + LANES, LANES)]
            i0 = iv[pl.ds(w, LANES)]
            i1 = iv[pl.ds(w + LANES, LANES)]
            n0 = nv[pl.ds(w, LANES)]
            n1 = nv[pl.ds(w + LANES, LANES)]
            tv = u0 * (i0 - n0) + u1 * (i1 - n1)
            acc = jnp.where(lane_iota == j, jnp.sum(tv), acc)
        prob = 1.0 / (1.0 + jnp.exp(-acc))
        out_v[pl.ds(g * LANES, LANES)] = prob
        return ()

    lax.fori_loop(0, BPW // LANES, group_body, ())
    pltpu.sync_copy(out_v, out_hbm.at[pl.ds(base, BPW)])


@jax.jit
def kernel(user_table, item_table, user_tensor, item_tensor, nega_item_tensor):
    mesh = plsc.VectorSubcoreMesh(core_axis_name="c", subcore_axis_name="s")
    params = pltpu.CompilerParams(
        needs_layout_passes=False, use_tc_tiling_on_sc=True,
        disable_bounds_checks=True)

    extract = pl.kernel(
        _extract_body,
        out_type=jax.ShapeDtypeStruct((3 * BATCH * EMBED,), jnp.float32),
        mesh=mesh,
        scratch_types=[
            pltpu.VMEM((3 * BATCH,), jnp.int32),
            pltpu.VMEM((3 * RCAP,), jnp.int32),
            pltpu.VMEM((3 * RCAP,), jnp.int32),
            pltpu.VMEM((3 * RCAP,), jnp.int32),
            pltpu.VMEM((3 * RCAP,), jnp.int32),
            pltpu.VMEM((DEPTH, EMBED, 128), jnp.float32),
            pltpu.VMEM((STAG, EMBED), jnp.float32),
            pltpu.SMEM((NBUCK + 1,), jnp.int32),
            pltpu.SMEM((NBUCK,), jnp.int32),
            pltpu.SemaphoreType.DMA,
            pltpu.SemaphoreType.DMA,
        ],
        compiler_params=params,
    )
    compute = pl.kernel(
        _compute_body,
        out_type=jax.ShapeDtypeStruct((BATCH,), jnp.float32),
        mesh=mesh,
        scratch_types=[
            pltpu.VMEM((BPW * EMBED,), jnp.float32),
            pltpu.VMEM((BPW * EMBED,), jnp.float32),
            pltpu.VMEM((BPW * EMBED,), jnp.float32),
            pltpu.VMEM((BPW,), jnp.float32),
            pltpu.SemaphoreType.DMA,
        ],
        compiler_params=params,
    )

    emb = extract(
        user_table.T,
        item_table.T,
        user_tensor.astype(jnp.int32),
        item_tensor.astype(jnp.int32),
        nega_item_tensor.astype(jnp.int32),
    )
    return compute(emb)


# speedup vs baseline: 1.0073x; 1.0013x over previous
import functools

import jax
import jax.numpy as jnp
from jax import lax
from jax.experimental import pallas as pl
from jax.experimental.pallas import tpu as pltpu
from jax.experimental.pallas import tpu_sc as plsc

BATCH = 16384
EMBED = 32
ROWS = 1000000
NUM_CORES = 2
NUM_SUBCORES = 16
LANES = 16
NUM_WORKERS = NUM_CORES * NUM_SUBCORES
BPW = BATCH // NUM_WORKERS
TCOLS = (ROWS + 127) // 128
TPW = (TCOLS + NUM_WORKERS - 1) // NUM_WORKERS
NBUCK = 2 * TPW
RCAP = 768
DEPTH = 8
STAG = 32


def _extract_body(u_tab, i_tab, u_idx_hbm, i_idx_hbm, n_idx_hbm,
                  emb,
                  vidx, mly_r, mly_id, srt_r, srt_id, ring, stag,
                  starts, cursor,
                  sem_slab, sem_out):
    wid = lax.axis_index("s") * NUM_CORES + lax.axis_index("c")
    lo = wid * TPW
    hi = jnp.minimum(lo + TPW, TCOLS)

    idx_hbms = (u_idx_hbm, i_idx_hbm, n_idx_hbm)
    lane_iota = lax.iota(jnp.int32, LANES)

    for t in range(3):
        pltpu.sync_copy(idx_hbms[t], vidx.at[pl.ds(t * BATCH, BATCH)])

    def compact_body(i, offs):
        newoffs = []
        for t in range(3):
            off = offs[t]
            v = vidx[pl.ds(t * BATCH + i * LANES, LANES)]
            tc = lax.shift_right_logical(v, 7)
            mine = (tc >= lo) & (tc < hi)
            pc = jnp.cumsum(mine.astype(jnp.int32))
            pos = off + pc - 1
            k = t * BATCH + i * LANES + lane_iota
            plsc.store_scatter(mly_r, [pos], v, mask=mine)
            plsc.store_scatter(mly_id, [pos], k, mask=mine)
            newoffs.append(off + pc[LANES - 1])
        return tuple(newoffs)
    offs = lax.fori_loop(0, BATCH // LANES, compact_body,
                         tuple(jnp.int32(t * RCAP) for t in range(3)))
    nrecs = tuple(offs[t] - t * RCAP for t in range(3))

    def zero_body(b, _):
        cursor[b] = jnp.int32(0)
        return ()
    lax.fori_loop(0, NBUCK, zero_body, ())

    def hist_body(q, _, base=0, cnt=None):
        rv = mly_r[pl.ds(base + q * LANES, LANES)]
        iv = mly_id[pl.ds(base + q * LANES, LANES)]
        rem = cnt - q * LANES
        bv = jnp.clip((lax.shift_right_logical(rv, 7) - lo) * 2
                      + jnp.minimum(lax.shift_right_logical(iv, 14), 1),
                      0, NBUCK - 1)
        for lane in range(LANES):
            @pl.when(rem > lane)
            def _():
                b = bv[lane]
                cursor[b] = cursor[b] + 1
        return ()
    for t in range(3):
        lax.fori_loop(0, (nrecs[t] + LANES - 1) // LANES,
                      functools.partial(hist_body, base=t * RCAP,
                                        cnt=nrecs[t]), ())

    def scan_body(b, acc):
        c = cursor[b]
        starts[b] = acc
        cursor[b] = acc
        return acc + c
    total = lax.fori_loop(0, NBUCK, scan_body, jnp.int32(0))
    starts[NBUCK] = total

    def place_body(q, _, base=0, cnt=None):
        rv = mly_r[pl.ds(base + q * LANES, LANES)]
        iv = mly_id[pl.ds(base + q * LANES, LANES)]
        rem = cnt - q * LANES
        bv = jnp.clip((lax.shift_right_logical(rv, 7) - lo) * 2
                      + jnp.minimum(lax.shift_right_logical(iv, 14), 1),
                      0, NBUCK - 1)
        mask = lane_iota < rem
        pos = jnp.zeros((LANES,), jnp.int32)
        for lane in range(LANES):
            b = bv[lane]
            ps = cursor[b]
            pos = jnp.where(lane_iota == lane, ps, pos)
            @pl.when(rem > lane)
            def _():
                cursor[b] = ps + 1
        plsc.store_scatter(srt_r, [pos], rv, mask=mask)
        plsc.store_scatter(srt_id, [pos], iv, mask=mask)
        return ()
    for t in range(3):
        lax.fori_loop(0, (nrecs[t] + LANES - 1) // LANES,
                      functools.partial(place_body, base=t * RCAP,
                                        cnt=nrecs[t]), ())

    rows_lo = lane_iota
    rows_hi = lane_iota + LANES

    def issue_slab(b):
        cnt = starts[b + 1] - starts[b]
        col = lo + lax.shift_right_logical(b, 1)
        c0 = pl.multiple_of(col * 128, 128)
        @pl.when((cnt > 0) & (b % 2 == 0))
        def _():
            pltpu.async_copy(u_tab.at[:, pl.ds(c0, 128)],
                             ring.at[b % DEPTH], sem_slab)
        @pl.when((cnt > 0) & (b % 2 == 1))
        def _():
            pltpu.async_copy(i_tab.at[:, pl.ds(c0, 128)],
                             ring.at[b % DEPTH], sem_slab)

    def drain_slab():
        pltpu.make_async_copy(u_tab.at[:, pl.ds(0, 128)],
                              ring.at[0], sem_slab).wait()

    def process_bucket(p, n_out):
        beg = starts[p]
        cnt = starts[p + 1] - beg
        slot = p % DEPTH

        def rec_body(q, n_out):
            off = beg + q * LANES
            rv = srt_r[pl.ds(off, LANES)]
            iv = srt_id[pl.ds(off, LANES)]
            rem = cnt - q * LANES
            for lane in range(LANES):
                valid = rem > lane
                so = n_out % STAG
                @pl.when(valid)
                def _():
                    c = jnp.full((LANES,), rv[lane] & 127, jnp.int32)
                    v0 = plsc.load_gather(ring.at[slot], [rows_lo, c])
                    v1 = plsc.load_gather(ring.at[slot], [rows_hi, c])
                    @pl.when(n_out >= STAG)
                    def _():
                        pltpu.make_async_copy(
                            emb.at[pl.ds(0, EMBED)], stag.at[0],
                            sem_out).wait()
                    stag[so, pl.ds(0, LANES)] = v0
                    stag[so, pl.ds(LANES, LANES)] = v1
                    pltpu.async_copy(
                        stag.at[so],
                        emb.at[pl.ds(iv[lane] * EMBED, EMBED)],
                        sem_out)
                n_out = n_out + jnp.where(valid, 1, 0)
            return n_out

        nq = (cnt + LANES - 1) // LANES
        return lax.fori_loop(0, nq, rec_body, n_out)

    def maybe_process(p, n_out):
        @pl.when(starts[p + 1] - starts[p] > 0)
        def _():
            drain_slab()
        return lax.cond(starts[p + 1] - starts[p] > 0,
                        lambda n: process_bucket(p, n),
                        lambda n: n,
                        n_out)

    def bucket_body(b, n_out):
        issue_slab(b)
        p = b - (DEPTH - 1)
        return lax.cond(p >= 0,
                        lambda n: maybe_process(jnp.maximum(p, 0), n),
                        lambda n: n,
                        n_out)

    n_out = lax.fori_loop(0, NBUCK, bucket_body, jnp.int32(0))

    def tail_body(b0, n_out):
        return maybe_process(NBUCK - (DEPTH - 1) + b0, n_out)
    n_out = lax.fori_loop(0, DEPTH - 1, tail_body, n_out)

    def outdrain_body(i, _):
        pltpu.make_async_copy(emb.at[pl.ds(0, EMBED)], stag.at[0],
                              sem_out).wait()
        return ()
    lax.fori_loop(0, jnp.minimum(n_out, STAG), outdrain_body, ())


def _compute_body(emb, out_hbm, uv, iv, nv, out_v, sem):
    wid = lax.axis_index("s") * NUM_CORES + lax.axis_index("c")
    base = wid * BPW
    pltpu.sync_copy(emb.at[pl.ds(base * EMBED, BPW * EMBED)], uv)
    pltpu.sync_copy(emb.at[pl.ds((BATCH + base) * EMBED, BPW * EMBED)], iv)
    pltpu.sync_copy(emb.at[pl.ds((2 * BATCH + base) * EMBED, BPW * EMBED)], nv)

    lane_iota = lax.iota(jnp.int32, LANES)

    def group_body(g, _):
        acc = jnp.zeros((LANES,), jnp.float32)
        for j in range(LANES):
            w = (g * LANES + j) * EMBED
            u0 = uv[pl.ds(w, LANES)]
            u1 = uv[pl.ds(w + LANES, LANES)]
            i0 = iv[pl.ds(w, LANES)]
            i1 = iv[pl.ds(w + LANES, LANES)]
            n0 = nv[pl.ds(w, LANES)]
            n1 = nv[pl.ds(w + LANES, LANES)]
            tv = u0 * (i0 - n0) + u1 * (i1 - n1)
            acc = jnp.where(lane_iota == j, jnp.sum(tv), acc)
        prob = 1.0 / (1.0 + jnp.exp(-acc))
        out_v[pl.ds(g * LANES, LANES)] = prob
        return ()

    lax.fori_loop(0, BPW // LANES, group_body, ())
    pltpu.sync_copy(out_v, out_hbm.at[pl.ds(base, BPW)])


@jax.jit
def kernel(user_table, item_table, user_tensor, item_tensor, nega_item_tensor):
    mesh = plsc.VectorSubcoreMesh(core_axis_name="c", subcore_axis_name="s")
    params = pltpu.CompilerParams(
        needs_layout_passes=False, use_tc_tiling_on_sc=True,
        disable_bounds_checks=True)

    extract = pl.kernel(
        _extract_body,
        out_type=jax.ShapeDtypeStruct((3 * BATCH * EMBED,), jnp.float32),
        mesh=mesh,
        scratch_types=[
            pltpu.VMEM((3 * BATCH,), jnp.int32),
            pltpu.VMEM((3 * RCAP,), jnp.int32),
            pltpu.VMEM((3 * RCAP,), jnp.int32),
            pltpu.VMEM((3 * RCAP,), jnp.int32),
            pltpu.VMEM((3 * RCAP,), jnp.int32),
            pltpu.VMEM((DEPTH, EMBED, 128), jnp.float32),
            pltpu.VMEM((STAG, EMBED), jnp.float32),
            pltpu.SMEM((NBUCK + 1,), jnp.int32),
            pltpu.SMEM((NBUCK,), jnp.int32),
            pltpu.SemaphoreType.DMA,
            pltpu.SemaphoreType.DMA,
        ],
        compiler_params=params,
    )
    compute = pl.kernel(
        _compute_body,
        out_type=jax.ShapeDtypeStruct((BATCH,), jnp.float32),
        mesh=mesh,
        scratch_types=[
            pltpu.VMEM((BPW * EMBED,), jnp.float32),
            pltpu.VMEM((BPW * EMBED,), jnp.float32),
            pltpu.VMEM((BPW * EMBED,), jnp.float32),
            pltpu.VMEM((BPW,), jnp.float32),
            pltpu.SemaphoreType.DMA,
        ],
        compiler_params=params,
    )

    emb = extract(
        user_table.T,
        item_table.T,
        user_tensor.astype(jnp.int32),
        item_tensor.astype(jnp.int32),
        nega_item_tensor.astype(jnp.int32),
    )
    return compute(emb)


# speedup vs baseline: 1.1810x; 1.1724x over previous
import functools

import jax
import jax.numpy as jnp
from jax import lax
from jax.experimental import pallas as pl
from jax.experimental.pallas import tpu as pltpu
from jax.experimental.pallas import tpu_sc as plsc

BATCH = 16384
EMBED = 32
ROWS = 1000000
NUM_CORES = 2
NUM_SUBCORES = 16
LANES = 16
NUM_WORKERS = NUM_CORES * NUM_SUBCORES
BPW = BATCH // NUM_WORKERS
TCOLS = (ROWS + 127) // 128
TPW = (TCOLS + NUM_WORKERS - 1) // NUM_WORKERS
NBUCK = 2 * TPW
CAP = 2048
DEPTH = 8
STAG = 32


def _extract_body(u_tab, i_tab, u_idx_hbm, i_idx_hbm, n_idx_hbm,
                  u_emb, i_emb, n_emb,
                  vidx, mly_r, mly_id, srt_r, srt_id, ring, stag,
                  starts, cursor,
                  sem_slab, sem_out):
    wid = lax.axis_index("s") * NUM_CORES + lax.axis_index("c")
    lo = wid * TPW
    hi = jnp.minimum(lo + TPW, TCOLS)

    idx_hbms = (u_idx_hbm, i_idx_hbm, n_idx_hbm)
    embs = (u_emb, i_emb, n_emb)
    lane_iota = lax.iota(jnp.int32, LANES)

    for t in range(3):
        pltpu.sync_copy(idx_hbms[t], vidx.at[pl.ds(t * BATCH, BATCH)])

    nrec = jnp.int32(0)
    for t in range(3):
        def body(i, off, t=t):
            v = vidx[pl.ds(t * BATCH + i * LANES, LANES)]
            tc = lax.shift_right_logical(v, 7)
            mine = (tc >= lo) & (tc < hi)
            pc = jnp.cumsum(mine.astype(jnp.int32))
            pos = off + pc - 1
            k = i * LANES + lane_iota
            plsc.store_scatter(mly_r, [pos], v, mask=mine)
            plsc.store_scatter(mly_id, [pos], k * 4 + t, mask=mine)
            return off + pc[LANES - 1]
        nrec = lax.fori_loop(0, BATCH // LANES, body, nrec)

    def zero_body(b, _):
        cursor[b] = jnp.int32(0)
        return ()
    lax.fori_loop(0, NBUCK, zero_body, ())

    def hist_body(q, _):
        rv = mly_r[pl.ds(q * LANES, LANES)]
        iv = mly_id[pl.ds(q * LANES, LANES)]
        rem = nrec - q * LANES
        bv = jnp.clip((lax.shift_right_logical(rv, 7) - lo) * 2
                      + jnp.minimum(iv & 3, 1), 0, NBUCK - 1)
        for lane in range(LANES):
            @pl.when(rem > lane)
            def _():
                b = bv[lane]
                cursor[b] = cursor[b] + 1
        return ()
    lax.fori_loop(0, (nrec + LANES - 1) // LANES, hist_body, ())

    def scan_body(b, acc):
        c = cursor[b]
        starts[b] = acc
        cursor[b] = acc
        return acc + c
    total = lax.fori_loop(0, NBUCK, scan_body, jnp.int32(0))
    starts[NBUCK] = total

    def place_body(q, _):
        rv = mly_r[pl.ds(q * LANES, LANES)]
        iv = mly_id[pl.ds(q * LANES, LANES)]
        rem = nrec - q * LANES
        bv = jnp.clip((lax.shift_right_logical(rv, 7) - lo) * 2
                      + jnp.minimum(iv & 3, 1), 0, NBUCK - 1)
        mask = lane_iota < rem
        pos = jnp.zeros((LANES,), jnp.int32)
        for lane in range(LANES):
            b = bv[lane]
            ps = cursor[b]
            pos = jnp.where(lane_iota == lane, ps, pos)
            @pl.when(rem > lane)
            def _():
                cursor[b] = ps + 1
        plsc.store_scatter(srt_r, [pos], rv, mask=mask)
        plsc.store_scatter(srt_id, [pos], iv, mask=mask)
        return ()
    lax.fori_loop(0, (nrec + LANES - 1) // LANES, place_body, ())

    rows_lo = lane_iota
    rows_hi = lane_iota + LANES

    def issue_slab(b):
        cnt = starts[b + 1] - starts[b]
        col = lo + lax.shift_right_logical(b, 1)
        c0 = pl.multiple_of(col * 128, 128)
        @pl.when((cnt > 0) & (b % 2 == 0))
        def _():
            pltpu.async_copy(u_tab.at[:, pl.ds(c0, 128)],
                             ring.at[b % DEPTH], sem_slab)
        @pl.when((cnt > 0) & (b % 2 == 1))
        def _():
            pltpu.async_copy(i_tab.at[:, pl.ds(c0, 128)],
                             ring.at[b % DEPTH], sem_slab)

    def drain_slab():
        pltpu.make_async_copy(u_tab.at[:, pl.ds(0, 128)],
                              ring.at[0], sem_slab).wait()

    def process_bucket(p, n_out):
        beg = starts[p]
        cnt = starts[p + 1] - beg
        slot = p % DEPTH

        def rec_body(q, n_out):
            off = beg + q * LANES
            rv = srt_r[pl.ds(off, LANES)]
            iv = srt_id[pl.ds(off, LANES)]
            rem = cnt - q * LANES
            for lane in range(LANES):
                valid = rem > lane
                so = n_out % STAG
                @pl.when(valid)
                def _():
                    c = jnp.full((LANES,), rv[lane] & 127, jnp.int32)
                    v0 = plsc.load_gather(ring.at[slot], [rows_lo, c])
                    v1 = plsc.load_gather(ring.at[slot], [rows_hi, c])
                    @pl.when(n_out >= STAG)
                    def _():
                        pltpu.make_async_copy(
                            u_emb.at[pl.ds(0, EMBED)], stag.at[0],
                            sem_out).wait()
                    stag[so, pl.ds(0, LANES)] = v0
                    stag[so, pl.ds(LANES, LANES)] = v1
                    kk = lax.shift_right_logical(iv[lane], 2)
                    tt = iv[lane] & 3
                    for t in range(3):
                        @pl.when(tt == t)
                        def _(t=t):
                            pltpu.async_copy(
                                stag.at[so],
                                embs[t].at[pl.ds(kk * EMBED, EMBED)],
                                sem_out)
                n_out = n_out + jnp.where(valid, 1, 0)
            return n_out

        nq = (cnt + LANES - 1) // LANES
        return lax.fori_loop(0, nq, rec_body, n_out)

    def maybe_process(p, n_out):
        @pl.when(starts[p + 1] - starts[p] > 0)
        def _():
            drain_slab()
        return lax.cond(starts[p + 1] - starts[p] > 0,
                        lambda n: process_bucket(p, n),
                        lambda n: n,
                        n_out)

    def bucket_body(b, n_out):
        issue_slab(b)
        p = b - (DEPTH - 1)
        return lax.cond(p >= 0,
                        lambda n: maybe_process(jnp.maximum(p, 0), n),
                        lambda n: n,
                        n_out)

    n_out = lax.fori_loop(0, NBUCK, bucket_body, jnp.int32(0))

    def tail_body(b0, n_out):
        return maybe_process(NBUCK - (DEPTH - 1) + b0, n_out)
    n_out = lax.fori_loop(0, DEPTH - 1, tail_body, n_out)

    def outdrain_body(i, _):
        pltpu.make_async_copy(u_emb.at[pl.ds(0, EMBED)], stag.at[0],
                              sem_out).wait()
        return ()
    lax.fori_loop(0, jnp.minimum(n_out, STAG), outdrain_body, ())


def _compute_body(u_emb, i_emb, n_emb, out_hbm, uv, iv, nv, out_v, sem):
    wid = lax.axis_index("s") * NUM_CORES + lax.axis_index("c")
    base = wid * BPW
    pltpu.sync_copy(u_emb.at[pl.ds(base * EMBED, BPW * EMBED)], uv)
    pltpu.sync_copy(i_emb.at[pl.ds(base * EMBED, BPW * EMBED)], iv)
    pltpu.sync_copy(n_emb.at[pl.ds(base * EMBED, BPW * EMBED)], nv)

    lane_iota = lax.iota(jnp.int32, LANES)

    def group_body(g, _):
        acc = jnp.zeros((LANES,), jnp.float32)
        for j in range(LANES):
            w = (g * LANES + j) * EMBED
            u0 = uv[pl.ds(w, LANES)]
            u1 = uv[pl.ds(w + LANES, LANES)]
            i0 = iv[pl.ds(w, LANES)]
            i1 = iv[pl.ds(w + LANES, LANES)]
            n0 = nv[pl.ds(w, LANES)]
            n1 = nv[pl.ds(w + LANES, LANES)]
            tv = u0 * (i0 - n0) + u1 * (i1 - n1)
            acc = jnp.where(lane_iota == j, jnp.sum(tv), acc)
        prob = 1.0 / (1.0 + jnp.exp(-acc))
        out_v[pl.ds(g * LANES, LANES)] = prob
        return ()

    lax.fori_loop(0, BPW // LANES, group_body, ())
    pltpu.sync_copy(out_v, out_hbm.at[pl.ds(base, BPW)])


@jax.jit
def kernel(user_table, item_table, user_tensor, item_tensor, nega_item_tensor):
    mesh = plsc.VectorSubcoreMesh(core_axis_name="c", subcore_axis_name="s")
    params = pltpu.CompilerParams(
        needs_layout_passes=False, use_tc_tiling_on_sc=True,
        disable_bounds_checks=True)

    extract = pl.kernel(
        _extract_body,
        out_type=(
            jax.ShapeDtypeStruct((BATCH * EMBED,), jnp.float32),
            jax.ShapeDtypeStruct((BATCH * EMBED,), jnp.float32),
            jax.ShapeDtypeStruct((BATCH * EMBED,), jnp.float32),
        ),
        mesh=mesh,
        scratch_types=[
            pltpu.VMEM((3 * BATCH,), jnp.int32),
            pltpu.VMEM((CAP,), jnp.int32),
            pltpu.VMEM((CAP,), jnp.int32),
            pltpu.VMEM((CAP,), jnp.int32),
            pltpu.VMEM((CAP,), jnp.int32),
            pltpu.VMEM((DEPTH, EMBED, 128), jnp.float32),
            pltpu.VMEM((STAG, EMBED), jnp.float32),
            pltpu.SMEM((NBUCK + 1,), jnp.int32),
            pltpu.SMEM((NBUCK,), jnp.int32),
            pltpu.SemaphoreType.DMA,
            pltpu.SemaphoreType.DMA,
        ],
        compiler_params=params,
    )
    compute = pl.kernel(
        _compute_body,
        out_type=jax.ShapeDtypeStruct((BATCH,), jnp.float32),
        mesh=mesh,
        scratch_types=[
            pltpu.VMEM((BPW * EMBED,), jnp.float32),
            pltpu.VMEM((BPW * EMBED,), jnp.float32),
            pltpu.VMEM((BPW * EMBED,), jnp.float32),
            pltpu.VMEM((BPW,), jnp.float32),
            pltpu.SemaphoreType.DMA,
        ],
        compiler_params=params,
    )

    u_emb, i_emb, n_emb = extract(
        user_table.T,
        item_table.T,
        user_tensor.astype(jnp.int32),
        item_tensor.astype(jnp.int32),
        nega_item_tensor.astype(jnp.int32),
    )
    return compute(u_emb, i_emb, n_emb)
